# baseline (speedup 1.0000x reference)
import functools

import jax
import jax.numpy as jnp
from jax import lax
from jax.experimental import pallas as pl
from jax.experimental.pallas import tpu as pltpu
from jax.experimental.pallas import tpu_sc as plsc

N = 100000
E = 1600000
NC = 2
NS = 16
NW = NC * NS
CH = 128
CPT = 392
EP = NW * CPT * CH
N2 = 102400
NPAD = N2
RPT = NPAD // NS

def _make_mesh(nc):
  return plsc.VectorSubcoreMesh(
      core_axis_name="c", subcore_axis_name="s", num_cores=nc, num_subcores=NS)


K = 4
NSETS = 2


def _zero_fill(zbuf):
  def body(i, carry):
    zbuf[i] = jnp.zeros((16,), jnp.float32)
    return carry
  lax.fori_loop(0, CH, body, None)


def _zero_acc_slice(zbuf, acc, base):
  assert RPT % CH == 0
  def body(k, carry):
    pltpu.sync_copy(zbuf, acc.at[pl.ds(base + k * CH, CH)])
    return carry
  lax.fori_loop(0, RPT // CH, body, None)


def _writeback_slice(acc, out_hbm, c, base):
  def body(k, carry):
    pltpu.sync_copy(acc.at[pl.ds(base + k * CH, CH)],
                    out_hbm.at[c, pl.ds(base + k * CH, CH)])
    return carry
  lax.fori_loop(0, RPT // CH, body, None)


def _build_agg(nc):
  nw = nc * NS
  cpt = EP // (nw * CH)

  np_ = cpt // K

  @functools.partial(
      pl.kernel,
      out_type=jax.ShapeDtypeStruct((nc, NPAD, 16), jnp.float32),
      mesh=_make_mesh(nc),
      scratch_types=[
          pltpu.VMEM((NSETS, K, CH), jnp.int32),
          pltpu.VMEM((NSETS, K, CH), jnp.int32),
          pltpu.VMEM((NSETS, K, CH, 16), jnp.float32),
          pltpu.VMEM((CH, 16), jnp.float32),
          pltpu.VMEM_SHARED((NPAD, 16), jnp.float32),
          pltpu.SemaphoreType.DMA,
          pltpu.SemaphoreType.DMA,
          pltpu.SemaphoreType.DMA,
      ],
      compiler_params=pltpu.CompilerParams(use_tc_tiling_on_sc=False),
  )
  def sc_agg(src_hbm, dst_hbm, tbl_hbm, out_hbm, src_v, dst_v, rows_v, zbuf,
             acc, isem, gsem, ssem):
    c = lax.axis_index("c")
    s = lax.axis_index("s")
    wid = c * NS + s
    base = s * RPT
    tbase = wid * cpt
    _zero_fill(zbuf)
    _zero_acc_slice(zbuf, acc, base)
    plsc.subcore_barrier()

    def idx_load(ph, st, i):
      g = tbase + ph * K + i
      pltpu.make_async_copy(src_hbm.at[g], src_v.at[st, i], isem).start()
      pltpu.make_async_copy(dst_hbm.at[g], dst_v.at[st, i], isem).start()

    def idx_wait(ph, st, i):
      g = tbase + ph * K + i
      pltpu.make_async_copy(src_hbm.at[g], src_v.at[st, i], isem).wait()
      pltpu.make_async_copy(dst_hbm.at[g], dst_v.at[st, i], isem).wait()

    def gather(st, i):
      return pltpu.make_async_copy(tbl_hbm.at[src_v.at[st, i]],
                                   rows_v.at[st, i], gsem)

    def scatter(st, i):
      return pltpu.make_async_copy(rows_v.at[st, i],
                                   acc.at[dst_v.at[st, i]], ssem)

    for i in range(K):
      idx_load(0, 0, i)
    for i in range(K):
      idx_wait(0, 0, i)
    for i in range(K):
      gather(0, i).start()

    def phase(p, carry):
      a = p % NSETS
      nxt = (p + 1) % NSETS

      @pl.when(p >= NSETS - 1)
      def _drain_old_scatters():
        for i in range(K):
          scatter(nxt, i).wait()

      @pl.when(p + 1 < np_)
      def _prefetch_idx():
        for i in range(K):
          idx_load(p + 1, nxt, i)

      for i in range(K):
        gather(a, i).wait()
      for i in range(K):
        scatter(a, i).start(add=True)

      @pl.when(p + 1 < np_)
      def _fire_next_gathers():
        for i in range(K):
          idx_wait(p + 1, nxt, i)
        for i in range(K):
          gather(nxt, i).start()
      return carry

    lax.fori_loop(0, np_, phase, None)
    for q in range(NSETS - 1):
      ph = np_ - (NSETS - 1) + q
      if ph >= 0:
        for i in range(K):
          scatter(ph % NSETS, i).wait()
    plsc.subcore_barrier()
    _writeback_slice(acc, out_hbm, c, base)

  return sc_agg


def _build_deg(nc):
  nw = nc * NS
  cpt = EP // (nw * CH)

  np_ = cpt // K

  @functools.partial(
      pl.kernel,
      out_type=jax.ShapeDtypeStruct((nc, NPAD, 16), jnp.float32),
      mesh=_make_mesh(nc),
      scratch_types=[
          pltpu.VMEM((2, K, CH), jnp.int32),
          pltpu.VMEM((CH, 16), jnp.float32),
          pltpu.VMEM((CH, 16), jnp.float32),
          pltpu.VMEM_SHARED((NPAD, 16), jnp.float32),
          pltpu.SemaphoreType.DMA,
          pltpu.SemaphoreType.DMA,
      ],
      compiler_params=pltpu.CompilerParams(use_tc_tiling_on_sc=False),
  )
  def sc_deg(dst_hbm, out_hbm, dst_v, ones_v, zbuf, acc, isem, ssem):
    c = lax.axis_index("c")
    s = lax.axis_index("s")
    wid = c * NS + s
    base = s * RPT
    tbase = wid * cpt
    _zero_fill(zbuf)
    _zero_acc_slice(zbuf, acc, base)

    def ones_body(i, carry):
      ones_v[i] = jnp.full((16,), 1.0, jnp.float32)
      return carry
    lax.fori_loop(0, CH, ones_body, None)
    plsc.subcore_barrier()

    def idx_load(ph, st, i):
      pltpu.make_async_copy(dst_hbm.at[tbase + ph * K + i],
                            dst_v.at[st, i], isem).start()

    def idx_wait(ph, st, i):
      pltpu.make_async_copy(dst_hbm.at[tbase + ph * K + i],
                            dst_v.at[st, i], isem).wait()

    def scatter(st, i):
      return pltpu.make_async_copy(ones_v, acc.at[dst_v.at[st, i]], ssem)

    for i in range(K):
      idx_load(0, 0, i)
    for i in range(K):
      idx_wait(0, 0, i)

    def phase(p, carry):
      a = p % 2
      b = (a + 1) % 2

      @pl.when(p >= 1)
      def _drain_prev_scatters():
        for i in range(K):
          scatter(b, i).wait()

      @pl.when(p + 1 < np_)
      def _prefetch_idx():
        for i in range(K):
          idx_load(p + 1, b, i)

      for i in range(K):
        scatter(a, i).start(add=True)

      @pl.when(p + 1 < np_)
      def _wait_next_idx():
        for i in range(K):
          idx_wait(p + 1, b, i)
      return carry

    lax.fori_loop(0, np_, phase, None)
    last = (np_ - 1) % 2
    for i in range(K):
      scatter(last, i).wait()
    plsc.subcore_barrier()
    _writeback_slice(acc, out_hbm, c, base)

  return sc_deg


_sc_agg = _build_agg(NC)
_sc_deg = _build_deg(NC)


NP8 = N2 // 8
BP = 256


def _tcA(x_ref, w1k_ref, degp_ref, g1_ref, dis_ref):
  deg = 1.0 + degp_ref[0] + degp_ref[1]
  dis = lax.rsqrt(deg)
  xb = x_ref[...]
  m = jnp.dot(xb[:, 0, :], w1k_ref[0], preferred_element_type=jnp.float32)
  for p in range(1, 8):
    m = m + jnp.dot(xb[:, p, :], w1k_ref[p],
                    preferred_element_type=jnp.float32)
  dis_ref[...] = dis
  g1_ref[...] = dis * m


def _tcB(s_ref, g1_ref, dis_ref, b1_ref, g2_ref):
  dis = dis_ref[...]
  y = dis * (s_ref[0] + s_ref[1] + g1_ref[...]) + b1_ref[...]
  g2_ref[...] = dis * jnp.maximum(y, 0.0)


def _tcC(s_ref, g2_ref, dis_ref, w2k_ref, b2_ref, o0, o1, o2, o3):
  dis = dis_ref[...]
  a = dis * (s_ref[0] + s_ref[1] + g2_ref[...])
  for j, o in enumerate((o0, o1, o2, o3)):
    y = jnp.dot(a, w2k_ref[j], preferred_element_type=jnp.float32) + b2_ref[j]
    o[...] = dis * jnp.maximum(y, 0.0)


def _tcD(s0, s1, s2, s3, g30, g31, g32, g33, dis_ref, w3k_ref, b3_ref,
         w4k_ref, g4_ref):
  dis = dis_ref[...]
  y3 = b3_ref[...]
  for j, (sj, gj) in enumerate(((s0, g30), (s1, g31), (s2, g32), (s3, g33))):
    aj = dis * (sj[0] + sj[1] + gj[...])
    y3 = y3 + jnp.dot(aj, w3k_ref[j], preferred_element_type=jnp.float32)
  h3 = jnp.maximum(y3, 0.0)
  g4_ref[...] = dis * jnp.dot(h3, w4k_ref[...],
                              preferred_element_type=jnp.float32)


def _tcE(s_ref, g4_ref, dis_ref, b4_ref, wck_ref, bc_ref, out_ref):
  dis = dis_ref[...]
  y4 = dis * (s_ref[0] + s_ref[1] + g4_ref[...]) + b4_ref[...]
  h4 = jnp.maximum(y4, 0.0)
  out_ref[...] = jnp.dot(h4, wck_ref[...],
                         preferred_element_type=jnp.float32) + bc_ref[...]


def _pk_spec(d=128):
  return pl.BlockSpec((BP, d), lambda i: (i, 0))


def _part_spec():
  return pl.BlockSpec((2, BP, 128), lambda i: (0, i, 0))


def _full_spec(shape):
  nd = len(shape)
  return pl.BlockSpec(shape, lambda i: (0,) * nd)


def _pk_out(d=128):
  return jax.ShapeDtypeStruct((NP8, d), jnp.float32)


def _kron8(w):
  return jnp.kron(jnp.eye(8, dtype=jnp.float32), w)


def _tile8(b):
  return jnp.tile(b, 8).reshape(1, -1)


def _as_tbl(g_packed):
  return g_packed.reshape(N2, 16)


def _as_pk(s_part):
  return s_part.reshape(s_part.shape[0], NPAD // 8, 128)


def kernel(x, edge_index, W1, b1, W2, b2, W3, b3, W4, b4, Wc, bc):
  src = edge_index[0]
  dst = edge_index[1]
  padn = EP - E
  srcp = jnp.concatenate([src, jnp.zeros((padn,), src.dtype)]).reshape(EP // CH, CH)
  dump = N + (jnp.arange(padn, dtype=dst.dtype) % (N2 - N))
  dstp = jnp.concatenate([dst, dump]).reshape(EP // CH, CH)
  x2 = jnp.concatenate(
      [x, jnp.zeros((N2 - N, x.shape[1]), x.dtype)]).reshape(NP8, 8, 128)

  degp = _as_pk(_sc_deg(dstp))

  g1, dis = pl.pallas_call(
      _tcA, grid=(50,),
      in_specs=[pl.BlockSpec((BP, 8, 128), lambda i: (i, 0, 0)),
                _full_spec((8, 128, 128)), _part_spec()],
      out_specs=[_pk_spec(), _pk_spec()],
      out_shape=[_pk_out(), _pk_out()],
  )(x2, _kron8(W1).reshape(8, 128, 128), degp)

  s1 = _as_pk(_sc_agg(srcp, dstp, _as_tbl(g1)))
  g2 = pl.pallas_call(
      _tcB, grid=(50,),
      in_specs=[_part_spec(), _pk_spec(), _pk_spec(), _full_spec((1, 128))],
      out_specs=_pk_spec(),
      out_shape=_pk_out(),
  )(s1, g1, dis, _tile8(b1))

  s2 = _as_pk(_sc_agg(srcp, dstp, _as_tbl(g2)))
  w2k = jnp.stack([_kron8(W2[:, 16 * j:16 * (j + 1)]) for j in range(4)])
  b2k = jnp.stack([jnp.tile(b2[16 * j:16 * (j + 1)], 8) for j in range(4)])
  g3 = pl.pallas_call(
      _tcC, grid=(50,),
      in_specs=[_part_spec(), _pk_spec(), _pk_spec(),
                _full_spec((4, 128, 128)), _full_spec((4, 128))],
      out_specs=[_pk_spec()] * 4,
      out_shape=[_pk_out()] * 4,
  )(s2, g2, dis, w2k, b2k)

  s3 = [_as_pk(_sc_agg(srcp, dstp, _as_tbl(g3j))) for g3j in g3]
  w3k = jnp.stack([_kron8(W3[16 * j:16 * (j + 1), :]) for j in range(4)])
  g4 = pl.pallas_call(
      _tcD, grid=(50,),
      in_specs=[_part_spec()] * 4 + [_pk_spec()] * 4
      + [_pk_spec(), _full_spec((4, 128, 8 * 64)), _full_spec((1, 8 * 64)),
         _full_spec((8 * 64, 128))],
      out_specs=_pk_spec(),
      out_shape=_pk_out(),
  )(*s3, *g3, dis, w3k, _tile8(b3), _kron8(W4))

  s4 = _as_pk(_sc_agg(srcp, dstp, _as_tbl(g4)))
  out_p = pl.pallas_call(
      _tcE, grid=(50,),
      in_specs=[_part_spec(), _pk_spec(), _pk_spec(),
                _full_spec((1, 128)), _full_spec((128, 8 * 40)),
                _full_spec((1, 8 * 40))],
      out_specs=_pk_spec(8 * 40),
      out_shape=_pk_out(8 * 40),
  )(s4, g4, dis, _tile8(b4), _kron8(Wc), _tile8(bc))
  return out_p[:N // 8].reshape(N, 40)


# speedup vs baseline: 1.0270x; 1.0270x over previous
import functools

import jax
import jax.numpy as jnp
from jax import lax
from jax.experimental import pallas as pl
from jax.experimental.pallas import tpu as pltpu
from jax.experimental.pallas import tpu_sc as plsc

N = 100000
E = 1600000
NC = 2
NS = 16
NW = NC * NS
CH = 128
CPT0 = 420
CPT1 = 364
EP = NS * (CPT0 + CPT1) * CH
N2 = 102400
NPAD = N2
RPT = NPAD // NS

def _make_mesh(nc):
  return plsc.VectorSubcoreMesh(
      core_axis_name="c", subcore_axis_name="s", num_cores=nc, num_subcores=NS)


K = 4
NSETS = 2


def _zero_fill(zbuf):
  def body(i, carry):
    zbuf[i] = jnp.zeros((16,), jnp.float32)
    return carry
  lax.fori_loop(0, CH, body, None)


def _zero_acc_slice(zbuf, acc, base):
  assert RPT % CH == 0
  def body(k, carry):
    pltpu.sync_copy(zbuf, acc.at[pl.ds(base + k * CH, CH)])
    return carry
  lax.fori_loop(0, RPT // CH, body, None)


def _writeback_slice(acc, out_hbm, c, base):
  def body(k, carry):
    pltpu.sync_copy(acc.at[pl.ds(base + k * CH, CH)],
                    out_hbm.at[c, pl.ds(base + k * CH, CH)])
    return carry
  lax.fori_loop(0, RPT // CH, body, None)


def _build_agg(nc):

  @functools.partial(
      pl.kernel,
      out_type=jax.ShapeDtypeStruct((nc, NPAD, 16), jnp.float32),
      mesh=_make_mesh(nc),
      scratch_types=[
          pltpu.VMEM((NSETS, K, CH), jnp.int32),
          pltpu.VMEM((NSETS, K, CH), jnp.int32),
          pltpu.VMEM((NSETS, K, CH, 16), jnp.float32),
          pltpu.VMEM((CH, 16), jnp.float32),
          pltpu.VMEM_SHARED((NPAD, 16), jnp.float32),
          pltpu.SemaphoreType.DMA,
          pltpu.SemaphoreType.DMA,
          pltpu.SemaphoreType.DMA,
      ],
      compiler_params=pltpu.CompilerParams(use_tc_tiling_on_sc=False),
  )
  def sc_agg(src_hbm, dst_hbm, tbl_hbm, out_hbm, src_v, dst_v, rows_v, zbuf,
             acc, isem, gsem, ssem):
    c = lax.axis_index("c")
    s = lax.axis_index("s")
    base = s * RPT
    cpt = jnp.where(c == 0, CPT0, CPT1)
    np_ = cpt // K
    tbase = c * NS * CPT0 + s * cpt
    _zero_fill(zbuf)
    _zero_acc_slice(zbuf, acc, base)
    plsc.subcore_barrier()

    def idx_load(ph, st, i):
      g = tbase + ph * K + i
      pltpu.make_async_copy(src_hbm.at[g], src_v.at[st, i], isem).start()
      pltpu.make_async_copy(dst_hbm.at[g], dst_v.at[st, i], isem).start()

    def idx_wait(ph, st, i):
      g = tbase + ph * K + i
      pltpu.make_async_copy(src_hbm.at[g], src_v.at[st, i], isem).wait()
      pltpu.make_async_copy(dst_hbm.at[g], dst_v.at[st, i], isem).wait()

    def gather(st, i):
      return pltpu.make_async_copy(tbl_hbm.at[src_v.at[st, i]],
                                   rows_v.at[st, i], gsem)

    def scatter(st, i):
      return pltpu.make_async_copy(rows_v.at[st, i],
                                   acc.at[dst_v.at[st, i]], ssem)

    for i in range(K):
      idx_load(0, 0, i)
    for i in range(K):
      idx_wait(0, 0, i)
    for i in range(K):
      gather(0, i).start()

    def phase(p, carry):
      a = p % NSETS
      nxt = (p + 1) % NSETS

      @pl.when(p >= NSETS - 1)
      def _drain_old_scatters():
        for i in range(K):
          scatter(nxt, i).wait()

      @pl.when(p + 1 < np_)
      def _prefetch_idx():
        for i in range(K):
          idx_load(p + 1, nxt, i)

      for i in range(K):
        gather(a, i).wait()
      for i in range(K):
        scatter(a, i).start(add=True)

      @pl.when(p + 1 < np_)
      def _fire_next_gathers():
        for i in range(K):
          idx_wait(p + 1, nxt, i)
        for i in range(K):
          gather(nxt, i).start()
      return carry

    lax.fori_loop(0, np_, phase, None)
    for q in range(NSETS - 1):
      ph = np_ - (NSETS - 1) + q
      for i in range(K):
        scatter(ph % NSETS, i).wait()
    plsc.subcore_barrier()
    _writeback_slice(acc, out_hbm, c, base)

  return sc_agg


def _build_deg(nc):

  @functools.partial(
      pl.kernel,
      out_type=jax.ShapeDtypeStruct((nc, NPAD, 16), jnp.float32),
      mesh=_make_mesh(nc),
      scratch_types=[
          pltpu.VMEM((2, K, CH), jnp.int32),
          pltpu.VMEM((CH, 16), jnp.float32),
          pltpu.VMEM((CH, 16), jnp.float32),
          pltpu.VMEM_SHARED((NPAD, 16), jnp.float32),
          pltpu.SemaphoreType.DMA,
          pltpu.SemaphoreType.DMA,
      ],
      compiler_params=pltpu.CompilerParams(use_tc_tiling_on_sc=False),
  )
  def sc_deg(dst_hbm, out_hbm, dst_v, ones_v, zbuf, acc, isem, ssem):
    c = lax.axis_index("c")
    s = lax.axis_index("s")
    base = s * RPT
    cpt = jnp.where(c == 0, CPT0, CPT1)
    np_ = cpt // K
    tbase = c * NS * CPT0 + s * cpt
    _zero_fill(zbuf)
    _zero_acc_slice(zbuf, acc, base)

    def ones_body(i, carry):
      ones_v[i] = jnp.full((16,), 1.0, jnp.float32)
      return carry
    lax.fori_loop(0, CH, ones_body, None)
    plsc.subcore_barrier()

    def idx_load(ph, st, i):
      pltpu.make_async_copy(dst_hbm.at[tbase + ph * K + i],
                            dst_v.at[st, i], isem).start()

    def idx_wait(ph, st, i):
      pltpu.make_async_copy(dst_hbm.at[tbase + ph * K + i],
                            dst_v.at[st, i], isem).wait()

    def scatter(st, i):
      return pltpu.make_async_copy(ones_v, acc.at[dst_v.at[st, i]], ssem)

    for i in range(K):
      idx_load(0, 0, i)
    for i in range(K):
      idx_wait(0, 0, i)

    def phase(p, carry):
      a = p % 2
      b = (a + 1) % 2

      @pl.when(p >= 1)
      def _drain_prev_scatters():
        for i in range(K):
          scatter(b, i).wait()

      @pl.when(p + 1 < np_)
      def _prefetch_idx():
        for i in range(K):
          idx_load(p + 1, b, i)

      for i in range(K):
        scatter(a, i).start(add=True)

      @pl.when(p + 1 < np_)
      def _wait_next_idx():
        for i in range(K):
          idx_wait(p + 1, b, i)
      return carry

    lax.fori_loop(0, np_, phase, None)
    last = (np_ - 1) % 2
    for i in range(K):
      scatter(last, i).wait()
    plsc.subcore_barrier()
    _writeback_slice(acc, out_hbm, c, base)

  return sc_deg


_sc_agg = _build_agg(NC)
_sc_deg = _build_deg(NC)


NP8 = N2 // 8
BP = 256


def _tcA(x_ref, w1k_ref, degp_ref, g1_ref, dis_ref):
  deg = 1.0 + degp_ref[0] + degp_ref[1]
  dis = lax.rsqrt(deg)
  xb = x_ref[...]
  m = jnp.dot(xb[:, 0, :], w1k_ref[0], preferred_element_type=jnp.float32)
  for p in range(1, 8):
    m = m + jnp.dot(xb[:, p, :], w1k_ref[p],
                    preferred_element_type=jnp.float32)
  dis_ref[...] = dis
  g1_ref[...] = dis * m


def _tcB(s_ref, g1_ref, dis_ref, b1_ref, g2_ref):
  dis = dis_ref[...]
  y = dis * (s_ref[0] + s_ref[1] + g1_ref[...]) + b1_ref[...]
  g2_ref[...] = dis * jnp.maximum(y, 0.0)


def _tcC(s_ref, g2_ref, dis_ref, w2k_ref, b2_ref, o0, o1, o2, o3):
  dis = dis_ref[...]
  a = dis * (s_ref[0] + s_ref[1] + g2_ref[...])
  for j, o in enumerate((o0, o1, o2, o3)):
    y = jnp.dot(a, w2k_ref[j], preferred_element_type=jnp.float32) + b2_ref[j]
    o[...] = dis * jnp.maximum(y, 0.0)


def _tcD(s0, s1, s2, s3, g30, g31, g32, g33, dis_ref, w3k_ref, b3_ref,
         w4k_ref, g4_ref):
  dis = dis_ref[...]
  y3 = b3_ref[...]
  for j, (sj, gj) in enumerate(((s0, g30), (s1, g31), (s2, g32), (s3, g33))):
    aj = dis * (sj[0] + sj[1] + gj[...])
    y3 = y3 + jnp.dot(aj, w3k_ref[j], preferred_element_type=jnp.float32)
  h3 = jnp.maximum(y3, 0.0)
  g4_ref[...] = dis * jnp.dot(h3, w4k_ref[...],
                              preferred_element_type=jnp.float32)


def _tcE(s_ref, g4_ref, dis_ref, b4_ref, wck_ref, bc_ref, out_ref):
  dis = dis_ref[...]
  y4 = dis * (s_ref[0] + s_ref[1] + g4_ref[...]) + b4_ref[...]
  h4 = jnp.maximum(y4, 0.0)
  out_ref[...] = jnp.dot(h4, wck_ref[...],
                         preferred_element_type=jnp.float32) + bc_ref[...]


def _pk_spec(d=128):
  return pl.BlockSpec((BP, d), lambda i: (i, 0))


def _part_spec():
  return pl.BlockSpec((2, BP, 128), lambda i: (0, i, 0))


def _full_spec(shape):
  nd = len(shape)
  return pl.BlockSpec(shape, lambda i: (0,) * nd)


def _pk_out(d=128):
  return jax.ShapeDtypeStruct((NP8, d), jnp.float32)


def _kron8(w):
  return jnp.kron(jnp.eye(8, dtype=jnp.float32), w)


def _tile8(b):
  return jnp.tile(b, 8).reshape(1, -1)


def _as_tbl(g_packed):
  return g_packed.reshape(N2, 16)


def _as_pk(s_part):
  return s_part.reshape(s_part.shape[0], NPAD // 8, 128)


def kernel(x, edge_index, W1, b1, W2, b2, W3, b3, W4, b4, Wc, bc):
  src = edge_index[0]
  dst = edge_index[1]
  padn = EP - E
  srcp = jnp.concatenate([src, jnp.zeros((padn,), src.dtype)]).reshape(EP // CH, CH)
  dump = N + (jnp.arange(padn, dtype=dst.dtype) % (N2 - N))
  dstp = jnp.concatenate([dst, dump]).reshape(EP // CH, CH)
  x2 = jnp.concatenate(
      [x, jnp.zeros((N2 - N, x.shape[1]), x.dtype)]).reshape(NP8, 8, 128)

  degp = _as_pk(_sc_deg(dstp))

  g1, dis = pl.pallas_call(
      _tcA, grid=(50,),
      in_specs=[pl.BlockSpec((BP, 8, 128), lambda i: (i, 0, 0)),
                _full_spec((8, 128, 128)), _part_spec()],
      out_specs=[_pk_spec(), _pk_spec()],
      out_shape=[_pk_out(), _pk_out()],
  )(x2, _kron8(W1).reshape(8, 128, 128), degp)

  s1 = _as_pk(_sc_agg(srcp, dstp, _as_tbl(g1)))
  g2 = pl.pallas_call(
      _tcB, grid=(50,),
      in_specs=[_part_spec(), _pk_spec(), _pk_spec(), _full_spec((1, 128))],
      out_specs=_pk_spec(),
      out_shape=_pk_out(),
  )(s1, g1, dis, _tile8(b1))

  s2 = _as_pk(_sc_agg(srcp, dstp, _as_tbl(g2)))
  w2k = jnp.stack([_kron8(W2[:, 16 * j:16 * (j + 1)]) for j in range(4)])
  b2k = jnp.stack([jnp.tile(b2[16 * j:16 * (j + 1)], 8) for j in range(4)])
  g3 = pl.pallas_call(
      _tcC, grid=(50,),
      in_specs=[_part_spec(), _pk_spec(), _pk_spec(),
                _full_spec((4, 128, 128)), _full_spec((4, 128))],
      out_specs=[_pk_spec()] * 4,
      out_shape=[_pk_out()] * 4,
  )(s2, g2, dis, w2k, b2k)

  s3 = [_as_pk(_sc_agg(srcp, dstp, _as_tbl(g3j))) for g3j in g3]
  w3k = jnp.stack([_kron8(W3[16 * j:16 * (j + 1), :]) for j in range(4)])
  g4 = pl.pallas_call(
      _tcD, grid=(50,),
      in_specs=[_part_spec()] * 4 + [_pk_spec()] * 4
      + [_pk_spec(), _full_spec((4, 128, 8 * 64)), _full_spec((1, 8 * 64)),
         _full_spec((8 * 64, 128))],
      out_specs=_pk_spec(),
      out_shape=_pk_out(),
  )(*s3, *g3, dis, w3k, _tile8(b3), _kron8(W4))

  s4 = _as_pk(_sc_agg(srcp, dstp, _as_tbl(g4)))
  out_p = pl.pallas_call(
      _tcE, grid=(50,),
      in_specs=[_part_spec(), _pk_spec(), _pk_spec(),
                _full_spec((1, 128)), _full_spec((128, 8 * 40)),
                _full_spec((1, 8 * 40))],
      out_specs=_pk_spec(8 * 40),
      out_shape=_pk_out(8 * 40),
  )(s4, g4, dis, _tile8(b4), _kron8(Wc), _tile8(bc))
  return out_p[:N // 8].reshape(N, 40)


# speedup vs baseline: 1.1996x; 1.1680x over previous
import functools

import jax
import jax.numpy as jnp
from jax import lax
from jax.experimental import pallas as pl
from jax.experimental.pallas import tpu as pltpu
from jax.experimental.pallas import tpu_sc as plsc

N = 100000
E = 1600000
NC = 2
NS = 16
NW = NC * NS
CH = 128
CPT0 = 420
CPT1 = 364
EP = NS * (CPT0 + CPT1) * CH
N2 = 102400
NPAD = N2
RPT = NPAD // NS

def _make_mesh(nc):
  return plsc.VectorSubcoreMesh(
      core_axis_name="c", subcore_axis_name="s", num_cores=nc, num_subcores=NS)


K = 4
NSETS = 2
RI = 3


def _zero_fill(zbuf):
  def body(i, carry):
    zbuf[i] = jnp.zeros((16,), jnp.float32)
    return carry
  lax.fori_loop(0, CH, body, None)


def _zero_acc_slice(zbuf, acc, base):
  assert RPT % CH == 0
  def body(k, carry):
    pltpu.sync_copy(zbuf, acc.at[pl.ds(base + k * CH, CH)])
    return carry
  lax.fori_loop(0, RPT // CH, body, None)


def _writeback_slice(acc, out_hbm, c, base):
  def body(k, carry):
    pltpu.sync_copy(acc.at[pl.ds(base + k * CH, CH)],
                    out_hbm.at[c, pl.ds(base + k * CH, CH)])
    return carry
  lax.fori_loop(0, RPT // CH, body, None)


def _build_agg(nc):

  @functools.partial(
      pl.kernel,
      out_type=jax.ShapeDtypeStruct((nc, NPAD, 16), jnp.float32),
      mesh=_make_mesh(nc),
      scratch_types=[
          pltpu.VMEM((RI, K, CH), jnp.int32),
          pltpu.VMEM((RI, K, CH), jnp.int32),
          pltpu.VMEM((NSETS, K, CH, 16), jnp.float32),
          pltpu.VMEM((CH, 16), jnp.float32),
          pltpu.VMEM_SHARED((NPAD, 16), jnp.float32),
          pltpu.SemaphoreType.DMA,
          pltpu.SemaphoreType.DMA((2,)),
          pltpu.SemaphoreType.DMA,
      ],
      compiler_params=pltpu.CompilerParams(use_tc_tiling_on_sc=False),
  )
  def sc_agg(src_hbm, dst_hbm, tbl_hbm, out_hbm, src_v, dst_v, rows_v, zbuf,
             acc, isem, gsem, ssem):
    c = lax.axis_index("c")
    s = lax.axis_index("s")
    base = s * RPT
    cpt = jnp.where(c == 0, CPT0, CPT1)
    np_ = cpt // K
    tbase = c * NS * CPT0 + s * cpt
    _zero_fill(zbuf)
    _zero_acc_slice(zbuf, acc, base)
    plsc.subcore_barrier()

    def idx_load(ph, i):
      g = tbase + ph * K + i
      st = ph % RI
      pltpu.make_async_copy(src_hbm.at[g], src_v.at[st, i], isem).start()
      pltpu.make_async_copy(dst_hbm.at[g], dst_v.at[st, i], isem).start()

    def idx_wait(ph, i):
      g = tbase + ph * K + i
      st = ph % RI
      pltpu.make_async_copy(src_hbm.at[g], src_v.at[st, i], isem).wait()
      pltpu.make_async_copy(dst_hbm.at[g], dst_v.at[st, i], isem).wait()

    def gather(ph, i):
      return pltpu.make_async_copy(tbl_hbm.at[src_v.at[ph % RI, i]],
                                   rows_v.at[ph % NSETS, i],
                                   gsem.at[ph % NSETS])

    def scatter(ph, i):
      return pltpu.make_async_copy(rows_v.at[ph % NSETS, i],
                                   acc.at[dst_v.at[ph % RI, i]], ssem)

    for i in range(K):
      idx_load(0, i)
    for i in range(K):
      idx_load(1, i)
    for i in range(K):
      idx_wait(0, i)
    for i in range(K):
      gather(0, i).start()

    def phase(p, carry):
      @pl.when(p >= 1)
      def _drain_old_scatters():
        for i in range(K):
          scatter(p - 1, i).wait()

      @pl.when(p + 1 < np_)
      def _fire_next_gathers():
        for i in range(K):
          idx_wait(p + 1, i)
        for i in range(K):
          gather(p + 1, i).start()

      @pl.when(p + 2 < np_)
      def _prefetch_idx():
        for i in range(K):
          idx_load(p + 2, i)

      for i in range(K):
        gather(p, i).wait()
      for i in range(K):
        scatter(p, i).start(add=True)
      return carry

    lax.fori_loop(0, np_, phase, None)
    for i in range(K):
      scatter(np_ - 1, i).wait()
    plsc.subcore_barrier()
    _writeback_slice(acc, out_hbm, c, base)

  return sc_agg


def _build_deg(nc):

  @functools.partial(
      pl.kernel,
      out_type=jax.ShapeDtypeStruct((nc, NPAD, 16), jnp.float32),
      mesh=_make_mesh(nc),
      scratch_types=[
          pltpu.VMEM((2, K, CH), jnp.int32),
          pltpu.VMEM((CH, 16), jnp.float32),
          pltpu.VMEM((CH, 16), jnp.float32),
          pltpu.VMEM_SHARED((NPAD, 16), jnp.float32),
          pltpu.SemaphoreType.DMA,
          pltpu.SemaphoreType.DMA,
      ],
      compiler_params=pltpu.CompilerParams(use_tc_tiling_on_sc=False),
  )
  def sc_deg(dst_hbm, out_hbm, dst_v, ones_v, zbuf, acc, isem, ssem):
    c = lax.axis_index("c")
    s = lax.axis_index("s")
    base = s * RPT
    cpt = jnp.where(c == 0, CPT0, CPT1)
    np_ = cpt // K
    tbase = c * NS * CPT0 + s * cpt
    _zero_fill(zbuf)
    _zero_acc_slice(zbuf, acc, base)

    def ones_body(i, carry):
      ones_v[i] = jnp.full((16,), 1.0, jnp.float32)
      return carry
    lax.fori_loop(0, CH, ones_body, None)
    plsc.subcore_barrier()

    def idx_load(ph, st, i):
      pltpu.make_async_copy(dst_hbm.at[tbase + ph * K + i],
                            dst_v.at[st, i], isem).start()

    def idx_wait(ph, st, i):
      pltpu.make_async_copy(dst_hbm.at[tbase + ph * K + i],
                            dst_v.at[st, i], isem).wait()

    def scatter(st, i):
      return pltpu.make_async_copy(ones_v, acc.at[dst_v.at[st, i]], ssem)

    for i in range(K):
      idx_load(0, 0, i)
    for i in range(K):
      idx_wait(0, 0, i)

    def phase(p, carry):
      a = p % 2
      b = (a + 1) % 2

      @pl.when(p >= 1)
      def _drain_prev_scatters():
        for i in range(K):
          scatter(b, i).wait()

      @pl.when(p + 1 < np_)
      def _prefetch_idx():
        for i in range(K):
          idx_load(p + 1, b, i)

      for i in range(K):
        scatter(a, i).start(add=True)

      @pl.when(p + 1 < np_)
      def _wait_next_idx():
        for i in range(K):
          idx_wait(p + 1, b, i)
      return carry

    lax.fori_loop(0, np_, phase, None)
    last = (np_ - 1) % 2
    for i in range(K):
      scatter(last, i).wait()
    plsc.subcore_barrier()
    _writeback_slice(acc, out_hbm, c, base)

  return sc_deg


_sc_agg = _build_agg(NC)
_sc_deg = _build_deg(NC)


NP8 = N2 // 8
BP = 256


def _tcA(x_ref, w1k_ref, degp_ref, g1_ref, dis_ref):
  deg = 1.0 + degp_ref[0] + degp_ref[1]
  dis = lax.rsqrt(deg)
  xb = x_ref[...]
  m = jnp.dot(xb[:, 0, :], w1k_ref[0], preferred_element_type=jnp.float32)
  for p in range(1, 8):
    m = m + jnp.dot(xb[:, p, :], w1k_ref[p],
                    preferred_element_type=jnp.float32)
  dis_ref[...] = dis
  g1_ref[...] = dis * m


def _tcB(s_ref, g1_ref, dis_ref, b1_ref, g2_ref):
  dis = dis_ref[...]
  y = dis * (s_ref[0] + s_ref[1] + g1_ref[...]) + b1_ref[...]
  g2_ref[...] = dis * jnp.maximum(y, 0.0)


def _tcC(s_ref, g2_ref, dis_ref, w2k_ref, b2_ref, o0, o1, o2, o3):
  dis = dis_ref[...]
  a = dis * (s_ref[0] + s_ref[1] + g2_ref[...])
  for j, o in enumerate((o0, o1, o2, o3)):
    y = jnp.dot(a, w2k_ref[j], preferred_element_type=jnp.float32) + b2_ref[j]
    o[...] = dis * jnp.maximum(y, 0.0)


def _tcD(s0, s1, s2, s3, g30, g31, g32, g33, dis_ref, w3k_ref, b3_ref,
         w4k_ref, g4_ref):
  dis = dis_ref[...]
  y3 = b3_ref[...]
  for j, (sj, gj) in enumerate(((s0, g30), (s1, g31), (s2, g32), (s3, g33))):
    aj = dis * (sj[0] + sj[1] + gj[...])
    y3 = y3 + jnp.dot(aj, w3k_ref[j], preferred_element_type=jnp.float32)
  h3 = jnp.maximum(y3, 0.0)
  g4_ref[...] = dis * jnp.dot(h3, w4k_ref[...],
                              preferred_element_type=jnp.float32)


def _tcE(s_ref, g4_ref, dis_ref, b4_ref, wck_ref, bc_ref, out_ref):
  dis = dis_ref[...]
  y4 = dis * (s_ref[0] + s_ref[1] + g4_ref[...]) + b4_ref[...]
  h4 = jnp.maximum(y4, 0.0)
  out_ref[...] = jnp.dot(h4, wck_ref[...],
                         preferred_element_type=jnp.float32) + bc_ref[...]


def _pk_spec(d=128):
  return pl.BlockSpec((BP, d), lambda i: (i, 0))


def _part_spec():
  return pl.BlockSpec((2, BP, 128), lambda i: (0, i, 0))


def _full_spec(shape):
  nd = len(shape)
  return pl.BlockSpec(shape, lambda i: (0,) * nd)


def _pk_out(d=128):
  return jax.ShapeDtypeStruct((NP8, d), jnp.float32)


def _kron8(w):
  return jnp.kron(jnp.eye(8, dtype=jnp.float32), w)


def _tile8(b):
  return jnp.tile(b, 8).reshape(1, -1)


def _as_tbl(g_packed):
  return g_packed.reshape(N2, 16)


def _as_pk(s_part):
  return s_part.reshape(s_part.shape[0], NPAD // 8, 128)


def kernel(x, edge_index, W1, b1, W2, b2, W3, b3, W4, b4, Wc, bc):
  src = edge_index[0]
  dst = edge_index[1]
  padn = EP - E
  srcp = jnp.concatenate([src, jnp.zeros((padn,), src.dtype)]).reshape(EP // CH, CH)
  dump = N + (jnp.arange(padn, dtype=dst.dtype) % (N2 - N))
  dstp = jnp.concatenate([dst, dump]).reshape(EP // CH, CH)
  x2 = jnp.concatenate(
      [x, jnp.zeros((N2 - N, x.shape[1]), x.dtype)]).reshape(NP8, 8, 128)

  degp = _as_pk(_sc_deg(dstp))

  g1, dis = pl.pallas_call(
      _tcA, grid=(50,),
      in_specs=[pl.BlockSpec((BP, 8, 128), lambda i: (i, 0, 0)),
                _full_spec((8, 128, 128)), _part_spec()],
      out_specs=[_pk_spec(), _pk_spec()],
      out_shape=[_pk_out(), _pk_out()],
  )(x2, _kron8(W1).reshape(8, 128, 128), degp)

  s1 = _as_pk(_sc_agg(srcp, dstp, _as_tbl(g1)))
  g2 = pl.pallas_call(
      _tcB, grid=(50,),
      in_specs=[_part_spec(), _pk_spec(), _pk_spec(), _full_spec((1, 128))],
      out_specs=_pk_spec(),
      out_shape=_pk_out(),
  )(s1, g1, dis, _tile8(b1))

  s2 = _as_pk(_sc_agg(srcp, dstp, _as_tbl(g2)))
  w2k = jnp.stack([_kron8(W2[:, 16 * j:16 * (j + 1)]) for j in range(4)])
  b2k = jnp.stack([jnp.tile(b2[16 * j:16 * (j + 1)], 8) for j in range(4)])
  g3 = pl.pallas_call(
      _tcC, grid=(50,),
      in_specs=[_part_spec(), _pk_spec(), _pk_spec(),
                _full_spec((4, 128, 128)), _full_spec((4, 128))],
      out_specs=[_pk_spec()] * 4,
      out_shape=[_pk_out()] * 4,
  )(s2, g2, dis, w2k, b2k)

  s3 = [_as_pk(_sc_agg(srcp, dstp, _as_tbl(g3j))) for g3j in g3]
  w3k = jnp.stack([_kron8(W3[16 * j:16 * (j + 1), :]) for j in range(4)])
  g4 = pl.pallas_call(
      _tcD, grid=(50,),
      in_specs=[_part_spec()] * 4 + [_pk_spec()] * 4
      + [_pk_spec(), _full_spec((4, 128, 8 * 64)), _full_spec((1, 8 * 64)),
         _full_spec((8 * 64, 128))],
      out_specs=_pk_spec(),
      out_shape=_pk_out(),
  )(*s3, *g3, dis, w3k, _tile8(b3), _kron8(W4))

  s4 = _as_pk(_sc_agg(srcp, dstp, _as_tbl(g4)))
  out_p = pl.pallas_call(
      _tcE, grid=(50,),
      in_specs=[_part_spec(), _pk_spec(), _pk_spec(),
                _full_spec((1, 128)), _full_spec((128, 8 * 40)),
                _full_spec((1, 8 * 40))],
      out_specs=_pk_spec(8 * 40),
      out_shape=_pk_out(8 * 40),
  )(s4, g4, dis, _tile8(b4), _kron8(Wc), _tile8(bc))
  return out_p[:N // 8].reshape(N, 40)


# speedup vs baseline: 1.2120x; 1.0103x over previous
import functools

import jax
import jax.numpy as jnp
from jax import lax
from jax.experimental import pallas as pl
from jax.experimental.pallas import tpu as pltpu
from jax.experimental.pallas import tpu_sc as plsc

N = 100000
E = 1600000
NC = 2
NS = 16
NW = NC * NS
CH = 128
CPT0 = 440
CPT1 = 344
EP = NS * (CPT0 + CPT1) * CH
N2 = 102400
NPAD = N2
RPT = NPAD // NS

def _make_mesh(nc):
  return plsc.VectorSubcoreMesh(
      core_axis_name="c", subcore_axis_name="s", num_cores=nc, num_subcores=NS)


K = 4
NSETS = 2
RI = 3


def _zero_fill(zbuf):
  def body(i, carry):
    zbuf[i] = jnp.zeros((16,), jnp.float32)
    return carry
  lax.fori_loop(0, CH, body, None)


def _zero_acc_slice(zbuf, acc, base):
  assert RPT % CH == 0
  def body(k, carry):
    pltpu.sync_copy(zbuf, acc.at[pl.ds(base + k * CH, CH)])
    return carry
  lax.fori_loop(0, RPT // CH, body, None)


def _writeback_slice(acc, out_hbm, c, base):
  def body(k, carry):
    pltpu.sync_copy(acc.at[pl.ds(base + k * CH, CH)],
                    out_hbm.at[c, pl.ds(base + k * CH, CH)])
    return carry
  lax.fori_loop(0, RPT // CH, body, None)


def _build_agg(nc):

  @functools.partial(
      pl.kernel,
      out_type=jax.ShapeDtypeStruct((nc, NPAD, 16), jnp.float32),
      mesh=_make_mesh(nc),
      scratch_types=[
          pltpu.VMEM((RI, K, CH), jnp.int32),
          pltpu.VMEM((RI, K, CH), jnp.int32),
          pltpu.VMEM((NSETS, K, CH, 16), jnp.float32),
          pltpu.VMEM((CH, 16), jnp.float32),
          pltpu.VMEM_SHARED((NPAD, 16), jnp.float32),
          pltpu.SemaphoreType.DMA,
          pltpu.SemaphoreType.DMA((2,)),
          pltpu.SemaphoreType.DMA,
      ],
      compiler_params=pltpu.CompilerParams(use_tc_tiling_on_sc=False),
  )
  def sc_agg(src_hbm, dst_hbm, tbl_hbm, out_hbm, src_v, dst_v, rows_v, zbuf,
             acc, isem, gsem, ssem):
    c = lax.axis_index("c")
    s = lax.axis_index("s")
    base = s * RPT
    cpt = jnp.where(c == 0, CPT0, CPT1)
    np_ = cpt // K
    tbase = c * NS * CPT0 + s * cpt
    _zero_fill(zbuf)
    _zero_acc_slice(zbuf, acc, base)
    plsc.subcore_barrier()

    def idx_load(ph, i):
      g = tbase + ph * K + i
      st = ph % RI
      pltpu.make_async_copy(src_hbm.at[g], src_v.at[st, i], isem).start()
      pltpu.make_async_copy(dst_hbm.at[g], dst_v.at[st, i], isem).start()

    def idx_wait(ph, i):
      g = tbase + ph * K + i
      st = ph % RI
      pltpu.make_async_copy(src_hbm.at[g], src_v.at[st, i], isem).wait()
      pltpu.make_async_copy(dst_hbm.at[g], dst_v.at[st, i], isem).wait()

    def gather(ph, i):
      return pltpu.make_async_copy(tbl_hbm.at[src_v.at[ph % RI, i]],
                                   rows_v.at[ph % NSETS, i],
                                   gsem.at[ph % NSETS])

    def scatter(ph, i):
      return pltpu.make_async_copy(rows_v.at[ph % NSETS, i],
                                   acc.at[dst_v.at[ph % RI, i]], ssem)

    for i in range(K):
      idx_load(0, i)
    for i in range(K):
      idx_load(1, i)
    for i in range(K):
      idx_wait(0, i)
    for i in range(K):
      gather(0, i).start()

    def phase(p, carry):
      @pl.when(p >= 1)
      def _drain_old_scatters():
        for i in range(K):
          scatter(p - 1, i).wait()

      @pl.when(p + 1 < np_)
      def _fire_next_gathers():
        for i in range(K):
          idx_wait(p + 1, i)
        for i in range(K):
          gather(p + 1, i).start()

      @pl.when(p + 2 < np_)
      def _prefetch_idx():
        for i in range(K):
          idx_load(p + 2, i)

      for i in range(K):
        gather(p, i).wait()
      for i in range(K):
        scatter(p, i).start(add=True)
      return carry

    lax.fori_loop(0, np_, phase, None)
    for i in range(K):
      scatter(np_ - 1, i).wait()
    plsc.subcore_barrier()
    _writeback_slice(acc, out_hbm, c, base)

  return sc_agg


def _build_deg(nc):

  @functools.partial(
      pl.kernel,
      out_type=jax.ShapeDtypeStruct((nc, NPAD, 16), jnp.float32),
      mesh=_make_mesh(nc),
      scratch_types=[
          pltpu.VMEM((2, K, CH), jnp.int32),
          pltpu.VMEM((CH, 16), jnp.float32),
          pltpu.VMEM((CH, 16), jnp.float32),
          pltpu.VMEM_SHARED((NPAD, 16), jnp.float32),
          pltpu.SemaphoreType.DMA,
          pltpu.SemaphoreType.DMA,
      ],
      compiler_params=pltpu.CompilerParams(use_tc_tiling_on_sc=False),
  )
  def sc_deg(dst_hbm, out_hbm, dst_v, ones_v, zbuf, acc, isem, ssem):
    c = lax.axis_index("c")
    s = lax.axis_index("s")
    base = s * RPT
    cpt = jnp.where(c == 0, CPT0, CPT1)
    np_ = cpt // K
    tbase = c * NS * CPT0 + s * cpt
    _zero_fill(zbuf)
    _zero_acc_slice(zbuf, acc, base)

    def ones_body(i, carry):
      ones_v[i] = jnp.full((16,), 1.0, jnp.float32)
      return carry
    lax.fori_loop(0, CH, ones_body, None)
    plsc.subcore_barrier()

    def idx_load(ph, st, i):
      pltpu.make_async_copy(dst_hbm.at[tbase + ph * K + i],
                            dst_v.at[st, i], isem).start()

    def idx_wait(ph, st, i):
      pltpu.make_async_copy(dst_hbm.at[tbase + ph * K + i],
                            dst_v.at[st, i], isem).wait()

    def scatter(st, i):
      return pltpu.make_async_copy(ones_v, acc.at[dst_v.at[st, i]], ssem)

    for i in range(K):
      idx_load(0, 0, i)
    for i in range(K):
      idx_wait(0, 0, i)

    def phase(p, carry):
      a = p % 2
      b = (a + 1) % 2

      @pl.when(p >= 1)
      def _drain_prev_scatters():
        for i in range(K):
          scatter(b, i).wait()

      @pl.when(p + 1 < np_)
      def _prefetch_idx():
        for i in range(K):
          idx_load(p + 1, b, i)

      for i in range(K):
        scatter(a, i).start(add=True)

      @pl.when(p + 1 < np_)
      def _wait_next_idx():
        for i in range(K):
          idx_wait(p + 1, b, i)
      return carry

    lax.fori_loop(0, np_, phase, None)
    last = (np_ - 1) % 2
    for i in range(K):
      scatter(last, i).wait()
    plsc.subcore_barrier()
    _writeback_slice(acc, out_hbm, c, base)

  return sc_deg


_sc_agg = _build_agg(NC)
_sc_deg = _build_deg(NC)


NP8 = N2 // 8
BP = 256


def _tcA(x_ref, w1k_ref, degp_ref, g1_ref, dis_ref):
  deg = 1.0 + degp_ref[0] + degp_ref[1]
  dis = lax.rsqrt(deg)
  xb = x_ref[...]
  m = jnp.dot(xb[:, 0, :], w1k_ref[0], preferred_element_type=jnp.float32)
  for p in range(1, 8):
    m = m + jnp.dot(xb[:, p, :], w1k_ref[p],
                    preferred_element_type=jnp.float32)
  dis_ref[...] = dis
  g1_ref[...] = dis * m


def _tcB(s_ref, g1_ref, dis_ref, b1_ref, g2_ref):
  dis = dis_ref[...]
  y = dis * (s_ref[0] + s_ref[1] + g1_ref[...]) + b1_ref[...]
  g2_ref[...] = dis * jnp.maximum(y, 0.0)


def _tcC(s_ref, g2_ref, dis_ref, w2k_ref, b2_ref, o0, o1, o2, o3):
  dis = dis_ref[...]
  a = dis * (s_ref[0] + s_ref[1] + g2_ref[...])
  for j, o in enumerate((o0, o1, o2, o3)):
    y = jnp.dot(a, w2k_ref[j], preferred_element_type=jnp.float32) + b2_ref[j]
    o[...] = dis * jnp.maximum(y, 0.0)


def _tcD(s0, s1, s2, s3, g30, g31, g32, g33, dis_ref, w3k_ref, b3_ref,
         w4k_ref, g4_ref):
  dis = dis_ref[...]
  y3 = b3_ref[...]
  for j, (sj, gj) in enumerate(((s0, g30), (s1, g31), (s2, g32), (s3, g33))):
    aj = dis * (sj[0] + sj[1] + gj[...])
    y3 = y3 + jnp.dot(aj, w3k_ref[j], preferred_element_type=jnp.float32)
  h3 = jnp.maximum(y3, 0.0)
  g4_ref[...] = dis * jnp.dot(h3, w4k_ref[...],
                              preferred_element_type=jnp.float32)


def _tcE(s_ref, g4_ref, dis_ref, b4_ref, wck_ref, bc_ref, out_ref):
  dis = dis_ref[...]
  y4 = dis * (s_ref[0] + s_ref[1] + g4_ref[...]) + b4_ref[...]
  h4 = jnp.maximum(y4, 0.0)
  out_ref[...] = jnp.dot(h4, wck_ref[...],
                         preferred_element_type=jnp.float32) + bc_ref[...]


def _pk_spec(d=128):
  return pl.BlockSpec((BP, d), lambda i: (i, 0))


def _part_spec():
  return pl.BlockSpec((2, BP, 128), lambda i: (0, i, 0))


def _full_spec(shape):
  nd = len(shape)
  return pl.BlockSpec(shape, lambda i: (0,) * nd)


def _pk_out(d=128):
  return jax.ShapeDtypeStruct((NP8, d), jnp.float32)


def _kron8(w):
  return jnp.kron(jnp.eye(8, dtype=jnp.float32), w)


def _tile8(b):
  return jnp.tile(b, 8).reshape(1, -1)


def _as_tbl(g_packed):
  return g_packed.reshape(N2, 16)


def _as_pk(s_part):
  return s_part.reshape(s_part.shape[0], NPAD // 8, 128)


def kernel(x, edge_index, W1, b1, W2, b2, W3, b3, W4, b4, Wc, bc):
  src = edge_index[0]
  dst = edge_index[1]
  padn = EP - E
  srcp = jnp.concatenate([src, jnp.zeros((padn,), src.dtype)]).reshape(EP // CH, CH)
  dump = N + (jnp.arange(padn, dtype=dst.dtype) % (N2 - N))
  dstp = jnp.concatenate([dst, dump]).reshape(EP // CH, CH)
  x2 = jnp.concatenate(
      [x, jnp.zeros((N2 - N, x.shape[1]), x.dtype)]).reshape(NP8, 8, 128)

  degp = _as_pk(_sc_deg(dstp))

  g1, dis = pl.pallas_call(
      _tcA, grid=(50,),
      in_specs=[pl.BlockSpec((BP, 8, 128), lambda i: (i, 0, 0)),
                _full_spec((8, 128, 128)), _part_spec()],
      out_specs=[_pk_spec(), _pk_spec()],
      out_shape=[_pk_out(), _pk_out()],
  )(x2, _kron8(W1).reshape(8, 128, 128), degp)

  s1 = _as_pk(_sc_agg(srcp, dstp, _as_tbl(g1)))
  g2 = pl.pallas_call(
      _tcB, grid=(50,),
      in_specs=[_part_spec(), _pk_spec(), _pk_spec(), _full_spec((1, 128))],
      out_specs=_pk_spec(),
      out_shape=_pk_out(),
  )(s1, g1, dis, _tile8(b1))

  s2 = _as_pk(_sc_agg(srcp, dstp, _as_tbl(g2)))
  w2k = jnp.stack([_kron8(W2[:, 16 * j:16 * (j + 1)]) for j in range(4)])
  b2k = jnp.stack([jnp.tile(b2[16 * j:16 * (j + 1)], 8) for j in range(4)])
  g3 = pl.pallas_call(
      _tcC, grid=(50,),
      in_specs=[_part_spec(), _pk_spec(), _pk_spec(),
                _full_spec((4, 128, 128)), _full_spec((4, 128))],
      out_specs=[_pk_spec()] * 4,
      out_shape=[_pk_out()] * 4,
  )(s2, g2, dis, w2k, b2k)

  s3 = [_as_pk(_sc_agg(srcp, dstp, _as_tbl(g3j))) for g3j in g3]
  w3k = jnp.stack([_kron8(W3[16 * j:16 * (j + 1), :]) for j in range(4)])
  g4 = pl.pallas_call(
      _tcD, grid=(50,),
      in_specs=[_part_spec()] * 4 + [_pk_spec()] * 4
      + [_pk_spec(), _full_spec((4, 128, 8 * 64)), _full_spec((1, 8 * 64)),
         _full_spec((8 * 64, 128))],
      out_specs=_pk_spec(),
      out_shape=_pk_out(),
  )(*s3, *g3, dis, w3k, _tile8(b3), _kron8(W4))

  s4 = _as_pk(_sc_agg(srcp, dstp, _as_tbl(g4)))
  out_p = pl.pallas_call(
      _tcE, grid=(50,),
      in_specs=[_part_spec(), _pk_spec(), _pk_spec(),
                _full_spec((1, 128)), _full_spec((128, 8 * 40)),
                _full_spec((1, 8 * 40))],
      out_specs=_pk_spec(8 * 40),
      out_shape=_pk_out(8 * 40),
  )(s4, g4, dis, _tile8(b4), _kron8(Wc), _tile8(bc))
  return out_p[:N // 8].reshape(N, 40)


# speedup vs baseline: 1.2647x; 1.0435x over previous
import functools

import jax
import jax.numpy as jnp
from jax import lax
from jax.experimental import pallas as pl
from jax.experimental.pallas import tpu as pltpu
from jax.experimental.pallas import tpu_sc as plsc

N = 100000
E = 1600000
NC = 2
NS = 16
NW = NC * NS
CH = 128
CPT0 = 440
CPT1 = 344
EP = NS * (CPT0 + CPT1) * CH
N2 = 102400
NPAD = N2
RPT = NPAD // NS

def _make_mesh(nc):
  return plsc.VectorSubcoreMesh(
      core_axis_name="c", subcore_axis_name="s", num_cores=nc, num_subcores=NS)


K = 4
NSETS = 2
RI = 3


def _zero_fill(zbuf):
  def body(i, carry):
    zbuf[i] = jnp.zeros((16,), jnp.float32)
    return carry
  lax.fori_loop(0, CH, body, None)


def _zero_acc_slice(zbuf, acc, base):
  assert RPT % CH == 0
  def body(k, carry):
    pltpu.sync_copy(zbuf, acc.at[pl.ds(base + k * CH, CH)])
    return carry
  lax.fori_loop(0, RPT // CH, body, None)


def _writeback_slice(acc, out_hbm, c, base):
  def body(k, carry):
    pltpu.sync_copy(acc.at[pl.ds(base + k * CH, CH)],
                    out_hbm.at[c, pl.ds(base + k * CH, CH)])
    return carry
  lax.fori_loop(0, RPT // CH, body, None)


def _build_agg(nc):

  @functools.partial(
      pl.kernel,
      out_type=jax.ShapeDtypeStruct((nc, NPAD, 16), jnp.float32),
      mesh=_make_mesh(nc),
      scratch_types=[
          pltpu.VMEM((RI, K, CH), jnp.int32),
          pltpu.VMEM((RI, K, CH), jnp.int32),
          pltpu.VMEM((NSETS, K, CH, 16), jnp.float32),
          pltpu.VMEM((CH, 16), jnp.float32),
          pltpu.VMEM_SHARED((NPAD, 16), jnp.float32),
          pltpu.SemaphoreType.DMA,
          pltpu.SemaphoreType.DMA((2,)),
          pltpu.SemaphoreType.DMA,
      ],
      compiler_params=pltpu.CompilerParams(use_tc_tiling_on_sc=False),
  )
  def sc_agg(src_hbm, dst_hbm, tbl_hbm, out_hbm, src_v, dst_v, rows_v, zbuf,
             acc, isem, gsem, ssem):
    c = lax.axis_index("c")
    s = lax.axis_index("s")
    base = s * RPT
    cpt = jnp.where(c == 0, CPT0, CPT1)
    np_ = cpt // K
    tbase = c * NS * CPT0 + s * cpt
    _zero_fill(zbuf)
    _zero_acc_slice(zbuf, acc, base)
    plsc.subcore_barrier()

    def idx_load(ph, i):
      g = tbase + ph * K + i
      st = ph % RI
      pltpu.make_async_copy(src_hbm.at[g], src_v.at[st, i], isem).start()
      pltpu.make_async_copy(dst_hbm.at[g], dst_v.at[st, i], isem).start()

    def idx_wait(ph, i):
      g = tbase + ph * K + i
      st = ph % RI
      pltpu.make_async_copy(src_hbm.at[g], src_v.at[st, i], isem).wait()
      pltpu.make_async_copy(dst_hbm.at[g], dst_v.at[st, i], isem).wait()

    def gather(ph, i):
      return pltpu.make_async_copy(tbl_hbm.at[src_v.at[ph % RI, i]],
                                   rows_v.at[ph % NSETS, i],
                                   gsem.at[ph % NSETS])

    def scatter(ph, i):
      return pltpu.make_async_copy(rows_v.at[ph % NSETS, i],
                                   acc.at[dst_v.at[ph % RI, i]], ssem)

    for i in range(K):
      idx_load(0, i)
    for i in range(K):
      idx_load(1, i)
    for i in range(K):
      idx_wait(0, i)
    for i in range(K):
      gather(0, i).start()

    def phase(p, carry):
      @pl.when(p >= 1)
      def _drain_old_scatters():
        for i in range(K):
          scatter(p - 1, i).wait()

      @pl.when(p + 1 < np_)
      def _fire_next_gathers():
        for i in range(K):
          idx_wait(p + 1, i)
        for i in range(K):
          gather(p + 1, i).start()

      @pl.when(p + 2 < np_)
      def _prefetch_idx():
        for i in range(K):
          idx_load(p + 2, i)

      for i in range(K):
        gather(p, i).wait()
      for i in range(K):
        scatter(p, i).start(add=True)
      return carry

    lax.fori_loop(0, np_, phase, None)
    for i in range(K):
      scatter(np_ - 1, i).wait()
    plsc.subcore_barrier()
    _writeback_slice(acc, out_hbm, c, base)

  return sc_agg


def _build_deg(nc):

  @functools.partial(
      pl.kernel,
      out_type=jax.ShapeDtypeStruct((nc, NPAD, 16), jnp.float32),
      mesh=_make_mesh(nc),
      scratch_types=[
          pltpu.VMEM((2, K, CH), jnp.int32),
          pltpu.VMEM((CH, 16), jnp.float32),
          pltpu.VMEM((CH, 16), jnp.float32),
          pltpu.VMEM_SHARED((NPAD, 16), jnp.float32),
          pltpu.SemaphoreType.DMA,
          pltpu.SemaphoreType.DMA,
      ],
      compiler_params=pltpu.CompilerParams(use_tc_tiling_on_sc=False),
  )
  def sc_deg(dst_hbm, out_hbm, dst_v, ones_v, zbuf, acc, isem, ssem):
    c = lax.axis_index("c")
    s = lax.axis_index("s")
    base = s * RPT
    cpt = jnp.where(c == 0, CPT0, CPT1)
    np_ = cpt // K
    tbase = c * NS * CPT0 + s * cpt
    _zero_fill(zbuf)
    _zero_acc_slice(zbuf, acc, base)

    def ones_body(i, carry):
      ones_v[i] = jnp.full((16,), 1.0, jnp.float32)
      return carry
    lax.fori_loop(0, CH, ones_body, None)
    plsc.subcore_barrier()

    def idx_load(ph, st, i):
      pltpu.make_async_copy(dst_hbm.at[tbase + ph * K + i],
                            dst_v.at[st, i], isem).start()

    def idx_wait(ph, st, i):
      pltpu.make_async_copy(dst_hbm.at[tbase + ph * K + i],
                            dst_v.at[st, i], isem).wait()

    def scatter(st, i):
      return pltpu.make_async_copy(ones_v, acc.at[dst_v.at[st, i]], ssem)

    for i in range(K):
      idx_load(0, 0, i)
    for i in range(K):
      idx_wait(0, 0, i)

    def phase(p, carry):
      a = p % 2
      b = (a + 1) % 2

      @pl.when(p >= 1)
      def _drain_prev_scatters():
        for i in range(K):
          scatter(b, i).wait()

      @pl.when(p + 1 < np_)
      def _prefetch_idx():
        for i in range(K):
          idx_load(p + 1, b, i)

      for i in range(K):
        scatter(a, i).start(add=True)

      @pl.when(p + 1 < np_)
      def _wait_next_idx():
        for i in range(K):
          idx_wait(p + 1, b, i)
      return carry

    lax.fori_loop(0, np_, phase, None)
    last = (np_ - 1) % 2
    for i in range(K):
      scatter(last, i).wait()
    plsc.subcore_barrier()
    _writeback_slice(acc, out_hbm, c, base)

  return sc_deg


_sc_agg = _build_agg(NC)
_sc_deg = _build_deg(NC)


NP8 = N2 // 8
BP = 512


def _tcA(x_ref, w1k_ref, degp_ref, g1_ref, dis_ref):
  deg = 1.0 + degp_ref[0] + degp_ref[1]
  dis = lax.rsqrt(deg)
  xb = x_ref[...]
  m = jnp.dot(xb[:, 0, :], w1k_ref[0], preferred_element_type=jnp.float32)
  for p in range(1, 8):
    m = m + jnp.dot(xb[:, p, :], w1k_ref[p],
                    preferred_element_type=jnp.float32)
  dis_ref[...] = dis
  g1_ref[...] = dis * m


def _tcB(s_ref, g1_ref, dis_ref, b1_ref, g2_ref):
  dis = dis_ref[...]
  y = dis * (s_ref[0] + s_ref[1] + g1_ref[...]) + b1_ref[...]
  g2_ref[...] = dis * jnp.maximum(y, 0.0)


def _tcC(s_ref, g2_ref, dis_ref, w2k_ref, b2_ref, o0, o1, o2, o3):
  dis = dis_ref[...]
  a = dis * (s_ref[0] + s_ref[1] + g2_ref[...])
  for j, o in enumerate((o0, o1, o2, o3)):
    y = jnp.dot(a, w2k_ref[j], preferred_element_type=jnp.float32) + b2_ref[j]
    o[...] = dis * jnp.maximum(y, 0.0)


def _tcD(s0, s1, s2, s3, g30, g31, g32, g33, dis_ref, w3k_ref, b3_ref,
         w4k_ref, g4_ref):
  dis = dis_ref[...]
  y3 = b3_ref[...]
  for j, (sj, gj) in enumerate(((s0, g30), (s1, g31), (s2, g32), (s3, g33))):
    aj = dis * (sj[0] + sj[1] + gj[...])
    y3 = y3 + jnp.dot(aj, w3k_ref[j], preferred_element_type=jnp.float32)
  h3 = jnp.maximum(y3, 0.0)
  g4_ref[...] = dis * jnp.dot(h3, w4k_ref[...],
                              preferred_element_type=jnp.float32)


def _tcE(s_ref, g4_ref, dis_ref, b4_ref, wck_ref, bc_ref, out_ref):
  dis = dis_ref[...]
  y4 = dis * (s_ref[0] + s_ref[1] + g4_ref[...]) + b4_ref[...]
  h4 = jnp.maximum(y4, 0.0)
  out_ref[...] = jnp.dot(h4, wck_ref[...],
                         preferred_element_type=jnp.float32) + bc_ref[...]


def _pk_spec(d=128):
  return pl.BlockSpec((BP, d), lambda i: (i, 0))


def _part_spec():
  return pl.BlockSpec((2, BP, 128), lambda i: (0, i, 0))


def _full_spec(shape):
  nd = len(shape)
  return pl.BlockSpec(shape, lambda i: (0,) * nd)


def _pk_out(d=128):
  return jax.ShapeDtypeStruct((NP8, d), jnp.float32)


def _kron8(w):
  return jnp.kron(jnp.eye(8, dtype=jnp.float32), w)


def _tile8(b):
  return jnp.tile(b, 8).reshape(1, -1)


def _as_tbl(g_packed):
  return g_packed.reshape(N2, 16)


def _as_pk(s_part):
  return s_part.reshape(s_part.shape[0], NPAD // 8, 128)


def kernel(x, edge_index, W1, b1, W2, b2, W3, b3, W4, b4, Wc, bc):
  src = edge_index[0]
  dst = edge_index[1]
  padn = EP - E
  srcp = jnp.concatenate([src, jnp.zeros((padn,), src.dtype)]).reshape(EP // CH, CH)
  dump = N + (jnp.arange(padn, dtype=dst.dtype) % (N2 - N))
  dstp = jnp.concatenate([dst, dump]).reshape(EP // CH, CH)
  x2 = jnp.concatenate(
      [x, jnp.zeros((N2 - N, x.shape[1]), x.dtype)]).reshape(NP8, 8, 128)

  degp = _as_pk(_sc_deg(dstp))

  g1, dis = pl.pallas_call(
      _tcA, grid=(25,),
      in_specs=[pl.BlockSpec((BP, 8, 128), lambda i: (i, 0, 0)),
                _full_spec((8, 128, 128)), _part_spec()],
      out_specs=[_pk_spec(), _pk_spec()],
      out_shape=[_pk_out(), _pk_out()],
  )(x2, _kron8(W1).reshape(8, 128, 128), degp)

  s1 = _as_pk(_sc_agg(srcp, dstp, _as_tbl(g1)))
  g2 = pl.pallas_call(
      _tcB, grid=(25,),
      in_specs=[_part_spec(), _pk_spec(), _pk_spec(), _full_spec((1, 128))],
      out_specs=_pk_spec(),
      out_shape=_pk_out(),
  )(s1, g1, dis, _tile8(b1))

  s2 = _as_pk(_sc_agg(srcp, dstp, _as_tbl(g2)))
  w2k = jnp.stack([_kron8(W2[:, 16 * j:16 * (j + 1)]) for j in range(4)])
  b2k = jnp.stack([jnp.tile(b2[16 * j:16 * (j + 1)], 8) for j in range(4)])
  g3 = pl.pallas_call(
      _tcC, grid=(25,),
      in_specs=[_part_spec(), _pk_spec(), _pk_spec(),
                _full_spec((4, 128, 128)), _full_spec((4, 128))],
      out_specs=[_pk_spec()] * 4,
      out_shape=[_pk_out()] * 4,
  )(s2, g2, dis, w2k, b2k)

  s3 = [_as_pk(_sc_agg(srcp, dstp, _as_tbl(g3j))) for g3j in g3]
  w3k = jnp.stack([_kron8(W3[16 * j:16 * (j + 1), :]) for j in range(4)])
  g4 = pl.pallas_call(
      _tcD, grid=(25,),
      in_specs=[_part_spec()] * 4 + [_pk_spec()] * 4
      + [_pk_spec(), _full_spec((4, 128, 8 * 64)), _full_spec((1, 8 * 64)),
         _full_spec((8 * 64, 128))],
      out_specs=_pk_spec(),
      out_shape=_pk_out(),
  )(*s3, *g3, dis, w3k, _tile8(b3), _kron8(W4))

  s4 = _as_pk(_sc_agg(srcp, dstp, _as_tbl(g4)))
  out_p = pl.pallas_call(
      _tcE, grid=(25,),
      in_specs=[_part_spec(), _pk_spec(), _pk_spec(),
                _full_spec((1, 128)), _full_spec((128, 8 * 40)),
                _full_spec((1, 8 * 40))],
      out_specs=_pk_spec(8 * 40),
      out_shape=_pk_out(8 * 40),
  )(s4, g4, dis, _tile8(b4), _kron8(Wc), _tile8(bc))
  return out_p[:N // 8].reshape(N, 40)


# speedup vs baseline: 1.3022x; 1.0296x over previous
import functools

import jax
import jax.numpy as jnp
from jax import lax
from jax.experimental import pallas as pl
from jax.experimental.pallas import tpu as pltpu
from jax.experimental.pallas import tpu_sc as plsc

N = 100000
E = 1600000
NC = 2
NS = 16
NW = NC * NS
CH = 128
CPT0 = 440
CPT1 = 344
EP = NS * (CPT0 + CPT1) * CH
N2 = 102400
NPAD = N2
RPT = NPAD // NS

def _make_mesh(nc):
  return plsc.VectorSubcoreMesh(
      core_axis_name="c", subcore_axis_name="s", num_cores=nc, num_subcores=NS)


K = 4
NSETS = 2
RI = 3


def _zero_fill(zbuf):
  def body(i, carry):
    zbuf[i] = jnp.zeros((16,), jnp.float32)
    return carry
  lax.fori_loop(0, CH, body, None)


def _zero_acc_slice(zbuf, acc, base):
  assert RPT % CH == 0
  def body(k, carry):
    pltpu.sync_copy(zbuf, acc.at[pl.ds(base + k * CH, CH)])
    return carry
  lax.fori_loop(0, RPT // CH, body, None)


def _writeback_slice(acc, out_hbm, c, base):
  def body(k, carry):
    pltpu.sync_copy(acc.at[pl.ds(base + k * CH, CH)],
                    out_hbm.at[c, pl.ds(base + k * CH, CH)])
    return carry
  lax.fori_loop(0, RPT // CH, body, None)


def _build_agg(nc):

  @functools.partial(
      pl.kernel,
      out_type=jax.ShapeDtypeStruct((nc, NPAD, 16), jnp.float32),
      mesh=_make_mesh(nc),
      scratch_types=[
          pltpu.VMEM((RI, K, CH), jnp.int32),
          pltpu.VMEM((RI, K, CH), jnp.int32),
          pltpu.VMEM((NSETS, K, CH, 16), jnp.float32),
          pltpu.VMEM((CH, 16), jnp.float32),
          pltpu.VMEM_SHARED((NPAD, 16), jnp.float32),
          pltpu.SemaphoreType.DMA,
          pltpu.SemaphoreType.DMA((2,)),
          pltpu.SemaphoreType.DMA,
      ],
      compiler_params=pltpu.CompilerParams(use_tc_tiling_on_sc=False),
  )
  def sc_agg(src_hbm, dst_hbm, tbl_hbm, out_hbm, src_v, dst_v, rows_v, zbuf,
             acc, isem, gsem, ssem):
    c = lax.axis_index("c")
    s = lax.axis_index("s")
    base = s * RPT
    cpt = jnp.where(c == 0, CPT0, CPT1)
    np_ = cpt // K
    tbase = c * NS * CPT0 + s * cpt
    _zero_fill(zbuf)
    _zero_acc_slice(zbuf, acc, base)
    plsc.subcore_barrier()

    def idx_load(ph, i):
      g = tbase + ph * K + i
      st = ph % RI
      pltpu.make_async_copy(src_hbm.at[g], src_v.at[st, i], isem).start()
      pltpu.make_async_copy(dst_hbm.at[g], dst_v.at[st, i], isem).start()

    def idx_wait(ph, i):
      g = tbase + ph * K + i
      st = ph % RI
      pltpu.make_async_copy(src_hbm.at[g], src_v.at[st, i], isem).wait()
      pltpu.make_async_copy(dst_hbm.at[g], dst_v.at[st, i], isem).wait()

    def gather(ph, i):
      return pltpu.make_async_copy(tbl_hbm.at[src_v.at[ph % RI, i]],
                                   rows_v.at[ph % NSETS, i],
                                   gsem.at[ph % NSETS])

    def scatter(ph, i):
      return pltpu.make_async_copy(rows_v.at[ph % NSETS, i],
                                   acc.at[dst_v.at[ph % RI, i]], ssem)

    for i in range(K):
      idx_load(0, i)
    for i in range(K):
      idx_load(1, i)
    for i in range(K):
      idx_wait(0, i)
    for i in range(K):
      gather(0, i).start()

    def phase(p, carry):
      @pl.when(p >= 1)
      def _drain_old_scatters():
        for i in range(K):
          scatter(p - 1, i).wait()

      @pl.when(p + 1 < np_)
      def _fire_next_gathers():
        for i in range(K):
          idx_wait(p + 1, i)
        for i in range(K):
          gather(p + 1, i).start()

      @pl.when(p + 2 < np_)
      def _prefetch_idx():
        for i in range(K):
          idx_load(p + 2, i)

      for i in range(K):
        gather(p, i).wait()
      for i in range(K):
        scatter(p, i).start(add=True)
      return carry

    lax.fori_loop(0, np_, phase, None)
    for i in range(K):
      scatter(np_ - 1, i).wait()
    plsc.subcore_barrier()
    _writeback_slice(acc, out_hbm, c, base)

  return sc_agg


def _build_deg(nc):

  @functools.partial(
      pl.kernel,
      out_type=jax.ShapeDtypeStruct((nc, NPAD, 16), jnp.float32),
      mesh=_make_mesh(nc),
      scratch_types=[
          pltpu.VMEM((2, K, CH), jnp.int32),
          pltpu.VMEM((CH, 16), jnp.float32),
          pltpu.VMEM((CH, 16), jnp.float32),
          pltpu.VMEM_SHARED((NPAD, 16), jnp.float32),
          pltpu.SemaphoreType.DMA,
          pltpu.SemaphoreType.DMA,
      ],
      compiler_params=pltpu.CompilerParams(use_tc_tiling_on_sc=False),
  )
  def sc_deg(dst_hbm, out_hbm, dst_v, ones_v, zbuf, acc, isem, ssem):
    c = lax.axis_index("c")
    s = lax.axis_index("s")
    base = s * RPT
    cpt = jnp.where(c == 0, CPT0, CPT1)
    np_ = cpt // K
    tbase = c * NS * CPT0 + s * cpt
    _zero_fill(zbuf)
    _zero_acc_slice(zbuf, acc, base)

    def ones_body(i, carry):
      ones_v[i] = jnp.full((16,), 1.0, jnp.float32)
      return carry
    lax.fori_loop(0, CH, ones_body, None)
    plsc.subcore_barrier()

    def idx_load(ph, st, i):
      pltpu.make_async_copy(dst_hbm.at[tbase + ph * K + i],
                            dst_v.at[st, i], isem).start()

    def idx_wait(ph, st, i):
      pltpu.make_async_copy(dst_hbm.at[tbase + ph * K + i],
                            dst_v.at[st, i], isem).wait()

    def scatter(st, i):
      return pltpu.make_async_copy(ones_v, acc.at[dst_v.at[st, i]], ssem)

    for i in range(K):
      idx_load(0, 0, i)
    for i in range(K):
      idx_wait(0, 0, i)

    def phase(p, carry):
      a = p % 2
      b = (a + 1) % 2

      @pl.when(p >= 1)
      def _drain_prev_scatters():
        for i in range(K):
          scatter(b, i).wait()

      @pl.when(p + 1 < np_)
      def _prefetch_idx():
        for i in range(K):
          idx_load(p + 1, b, i)

      for i in range(K):
        scatter(a, i).start(add=True)

      @pl.when(p + 1 < np_)
      def _wait_next_idx():
        for i in range(K):
          idx_wait(p + 1, b, i)
      return carry

    lax.fori_loop(0, np_, phase, None)
    last = (np_ - 1) % 2
    for i in range(K):
      scatter(last, i).wait()
    plsc.subcore_barrier()
    _writeback_slice(acc, out_hbm, c, base)

  return sc_deg


_sc_agg = _build_agg(NC)
_sc_deg = _build_deg(NC)


NP8 = N2 // 8
BP = 1280


def _tcA(x_ref, w1k_ref, degp_ref, g1_ref, dis_ref):
  deg = 1.0 + degp_ref[0] + degp_ref[1]
  dis = lax.rsqrt(deg)
  xb = x_ref[...]
  m = jnp.dot(xb[:, 0, :], w1k_ref[0], preferred_element_type=jnp.float32)
  for p in range(1, 8):
    m = m + jnp.dot(xb[:, p, :], w1k_ref[p],
                    preferred_element_type=jnp.float32)
  dis_ref[...] = dis
  g1_ref[...] = dis * m


def _tcB(s_ref, g1_ref, dis_ref, b1_ref, g2_ref):
  dis = dis_ref[...]
  y = dis * (s_ref[0] + s_ref[1] + g1_ref[...]) + b1_ref[...]
  g2_ref[...] = dis * jnp.maximum(y, 0.0)


def _tcC(s_ref, g2_ref, dis_ref, w2k_ref, b2_ref, o0, o1, o2, o3):
  dis = dis_ref[...]
  a = dis * (s_ref[0] + s_ref[1] + g2_ref[...])
  for j, o in enumerate((o0, o1, o2, o3)):
    y = jnp.dot(a, w2k_ref[j], preferred_element_type=jnp.float32) + b2_ref[j]
    o[...] = dis * jnp.maximum(y, 0.0)


def _tcD(s0, s1, s2, s3, g30, g31, g32, g33, dis_ref, w3k_ref, b3_ref,
         w4k_ref, g4_ref):
  dis = dis_ref[...]
  y3 = b3_ref[...]
  for j, (sj, gj) in enumerate(((s0, g30), (s1, g31), (s2, g32), (s3, g33))):
    aj = dis * (sj[0] + sj[1] + gj[...])
    y3 = y3 + jnp.dot(aj, w3k_ref[j], preferred_element_type=jnp.float32)
  h3 = jnp.maximum(y3, 0.0)
  g4_ref[...] = dis * jnp.dot(h3, w4k_ref[...],
                              preferred_element_type=jnp.float32)


def _tcE(s_ref, g4_ref, dis_ref, b4_ref, wck_ref, bc_ref, out_ref):
  dis = dis_ref[...]
  y4 = dis * (s_ref[0] + s_ref[1] + g4_ref[...]) + b4_ref[...]
  h4 = jnp.maximum(y4, 0.0)
  out_ref[...] = jnp.dot(h4, wck_ref[...],
                         preferred_element_type=jnp.float32) + bc_ref[...]


def _pk_spec(d=128):
  return pl.BlockSpec((BP, d), lambda i: (i, 0))


def _part_spec():
  return pl.BlockSpec((2, BP, 128), lambda i: (0, i, 0))


def _full_spec(shape):
  nd = len(shape)
  return pl.BlockSpec(shape, lambda i: (0,) * nd)


def _pk_out(d=128):
  return jax.ShapeDtypeStruct((NP8, d), jnp.float32)


def _kron8(w):
  return jnp.kron(jnp.eye(8, dtype=jnp.float32), w)


def _tile8(b):
  return jnp.tile(b, 8).reshape(1, -1)


def _as_tbl(g_packed):
  return g_packed.reshape(N2, 16)


def _as_pk(s_part):
  return s_part.reshape(s_part.shape[0], NPAD // 8, 128)


def kernel(x, edge_index, W1, b1, W2, b2, W3, b3, W4, b4, Wc, bc):
  src = edge_index[0]
  dst = edge_index[1]
  padn = EP - E
  srcp = jnp.concatenate([src, jnp.zeros((padn,), src.dtype)]).reshape(EP // CH, CH)
  dump = N + (jnp.arange(padn, dtype=dst.dtype) % (N2 - N))
  dstp = jnp.concatenate([dst, dump]).reshape(EP // CH, CH)
  x2 = jnp.concatenate(
      [x, jnp.zeros((N2 - N, x.shape[1]), x.dtype)]).reshape(NP8, 8, 128)

  degp = _as_pk(_sc_deg(dstp))

  g1, dis = pl.pallas_call(
      _tcA, grid=(10,),
      in_specs=[pl.BlockSpec((BP, 8, 128), lambda i: (i, 0, 0)),
                _full_spec((8, 128, 128)), _part_spec()],
      out_specs=[_pk_spec(), _pk_spec()],
      out_shape=[_pk_out(), _pk_out()],
  )(x2, _kron8(W1).reshape(8, 128, 128), degp)

  s1 = _as_pk(_sc_agg(srcp, dstp, _as_tbl(g1)))
  g2 = pl.pallas_call(
      _tcB, grid=(10,),
      in_specs=[_part_spec(), _pk_spec(), _pk_spec(), _full_spec((1, 128))],
      out_specs=_pk_spec(),
      out_shape=_pk_out(),
  )(s1, g1, dis, _tile8(b1))

  s2 = _as_pk(_sc_agg(srcp, dstp, _as_tbl(g2)))
  w2k = jnp.stack([_kron8(W2[:, 16 * j:16 * (j + 1)]) for j in range(4)])
  b2k = jnp.stack([jnp.tile(b2[16 * j:16 * (j + 1)], 8) for j in range(4)])
  g3 = pl.pallas_call(
      _tcC, grid=(10,),
      in_specs=[_part_spec(), _pk_spec(), _pk_spec(),
                _full_spec((4, 128, 128)), _full_spec((4, 128))],
      out_specs=[_pk_spec()] * 4,
      out_shape=[_pk_out()] * 4,
  )(s2, g2, dis, w2k, b2k)

  s3 = [_as_pk(_sc_agg(srcp, dstp, _as_tbl(g3j))) for g3j in g3]
  w3k = jnp.stack([_kron8(W3[16 * j:16 * (j + 1), :]) for j in range(4)])
  g4 = pl.pallas_call(
      _tcD, grid=(10,),
      in_specs=[_part_spec()] * 4 + [_pk_spec()] * 4
      + [_pk_spec(), _full_spec((4, 128, 8 * 64)), _full_spec((1, 8 * 64)),
         _full_spec((8 * 64, 128))],
      out_specs=_pk_spec(),
      out_shape=_pk_out(),
  )(*s3, *g3, dis, w3k, _tile8(b3), _kron8(W4))

  s4 = _as_pk(_sc_agg(srcp, dstp, _as_tbl(g4)))
  out_p = pl.pallas_call(
      _tcE, grid=(10,),
      in_specs=[_part_spec(), _pk_spec(), _pk_spec(),
                _full_spec((1, 128)), _full_spec((128, 8 * 40)),
                _full_spec((1, 8 * 40))],
      out_specs=_pk_spec(8 * 40),
      out_shape=_pk_out(8 * 40),
  )(s4, g4, dis, _tile8(b4), _kron8(Wc), _tile8(bc))
  return out_p[:N // 8].reshape(N, 40)
